# baseline (speedup 1.0000x reference)
import functools

import jax
import jax.numpy as jnp
from jax import lax
from jax.experimental import pallas as pl
from jax.experimental.pallas import tpu as pltpu
from jax.experimental.pallas import tpu_sc as plsc

VOCAB = 1000000
DIM = 64
B = 4096
L = 200
NLAB = 1000

NC = 2
NS = 16
NW = NC * NS
DOCS_PER_TILE = B // NW
TOK_PER_TILE = DOCS_PER_TILE * L
WDIM = 2 * DIM

CH0 = 128
CH1 = L - CH0


def _sc_segsum(doc_flat, table):
    mesh = plsc.VectorSubcoreMesh(core_axis_name="c", subcore_axis_name="s")

    @functools.partial(
        pl.kernel,
        mesh=mesh,
        out_type=jax.ShapeDtypeStruct((B * DIM,), jnp.float32),
        compiler_params=pltpu.CompilerParams(
            use_tc_tiling_on_sc=True, needs_layout_passes=False),
        scratch_types=[
            pltpu.VMEM((TOK_PER_TILE + 16,), jnp.int32),
            pltpu.VMEM((2, L, WDIM), jnp.float32),
            pltpu.VMEM((DOCS_PER_TILE * DIM,), jnp.float32),
            pltpu.SemaphoreType.DMA,
            pltpu.SemaphoreType.DMA,
        ],
    )
    def segsum(doc_hbm, tab_hbm, out_hbm, idx_v, rows_v, acc_v, s0, s1):
        wid = lax.axis_index("s") * NC + lax.axis_index("c")
        base = wid * TOK_PER_TILE
        sems = (s0, s1)

        pltpu.sync_copy(doc_hbm.at[pl.ds(base, TOK_PER_TILE)],
                        idx_v.at[pl.ds(0, TOK_PER_TILE)])

        def gathers(b, buf):
            sem = sems[buf]
            return (
                pltpu.make_async_copy(
                    tab_hbm.at[idx_v.at[pl.ds(b * L, CH0)]],
                    rows_v.at[buf, pl.ds(0, CH0)], sem),
                pltpu.make_async_copy(
                    tab_hbm.at[idx_v.at[pl.ds(b * L + CH0, CH1)]],
                    rows_v.at[buf, pl.ds(CH0, CH1)], sem),
            )

        def issue(b, buf):
            for g in gathers(b, buf):
                g.start()

        def drain(b, buf):
            for g in gathers(b, buf):
                g.wait()

        issue(0, 0)

        def per_doc(bb, _):
            for phase in range(2):
                b = 2 * bb + phase
                buf = phase

                @pl.when(b + 1 < DOCS_PER_TILE)
                def _prefetch():
                    issue(b + 1, 1 - buf)

                drain(b, buf)

                zero = jnp.zeros((16,), jnp.float32)

                def tok(t, accs):
                    new = []
                    for d in range(4):
                        new.append(accs[d] + rows_v[buf, t, pl.ds(16 * d, 16)])
                    return tuple(new)

                accs = lax.fori_loop(0, L, tok, (zero,) * 4)
                for d in range(4):
                    acc_v[pl.ds(b * DIM + 16 * d, 16)] = accs[d]
            return _

        lax.fori_loop(0, DOCS_PER_TILE // 2, per_doc, 0)
        pltpu.sync_copy(
            acc_v,
            out_hbm.at[pl.ds(wid * DOCS_PER_TILE * DIM, DOCS_PER_TILE * DIM)])

    return segsum(doc_flat, table)


def _tc_body(acc_ref, doc_ref, row0_ref, wd_ref, bd_ref, out_ref):
    npad = jnp.sum((doc_ref[...] == 0).astype(jnp.float32), axis=1,
                   keepdims=True)
    cnt = jnp.maximum(float(L) - npad, 1.0)
    enc = (acc_ref[...] - npad * row0_ref[...]) / cnt
    out_ref[...] = jnp.dot(enc, wd_ref[...],
                           preferred_element_type=jnp.float32) + bd_ref[...]


def _tc_decode(acc, doc, row0, Wd, bd2):
    bm = 512
    grid = B // bm
    return pl.pallas_call(
        _tc_body,
        grid=(grid,),
        in_specs=[
            pl.BlockSpec((bm, DIM), lambda i: (i, 0)),
            pl.BlockSpec((bm, L), lambda i: (i, 0)),
            pl.BlockSpec((1, DIM), lambda i: (0, 0)),
            pl.BlockSpec((DIM, NLAB), lambda i: (0, 0)),
            pl.BlockSpec((1, NLAB), lambda i: (0, 0)),
        ],
        out_specs=pl.BlockSpec((bm, NLAB), lambda i: (i, 0)),
        out_shape=jax.ShapeDtypeStruct((B, NLAB), jnp.float32),
    )(acc, doc, row0, Wd, bd2)


def kernel(doc, table, Wd, bd):
    tabw = jnp.concatenate([table, table], axis=1)
    acc_flat = _sc_segsum(doc.reshape(B * L), tabw)
    acc = acc_flat.reshape(B, DIM)
    row0 = lax.slice(table, (0, 0), (1, DIM))
    return _tc_decode(acc, doc, row0, Wd, bd.reshape(1, NLAB))


# speedup vs baseline: 1.5021x; 1.5021x over previous
import functools

import jax
import jax.numpy as jnp
from jax import lax
from jax.experimental import pallas as pl
from jax.experimental.pallas import tpu as pltpu
from jax.experimental.pallas import tpu_sc as plsc

VOCAB = 1000000
DIM = 64
B = 4096
L = 200
NLAB = 1000

NC = 2
NS = 16
NW = NC * NS
DOCS_PER_TILE = B // NW
TOK_PER_TILE = DOCS_PER_TILE * L
WDIM = 2 * DIM

CH0 = 128
CH1 = L - CH0


def _sc_segsum(doc_flat, table):
    mesh = plsc.VectorSubcoreMesh(core_axis_name="c", subcore_axis_name="s")

    @functools.partial(
        pl.kernel,
        mesh=mesh,
        out_type=jax.ShapeDtypeStruct((B * DIM,), jnp.float32),
        compiler_params=pltpu.CompilerParams(use_tc_tiling_on_sc=False),
        scratch_types=[
            pltpu.VMEM((TOK_PER_TILE + 16,), jnp.int32),
            pltpu.VMEM((2, L, DIM), jnp.float32),
            pltpu.VMEM((DOCS_PER_TILE * DIM,), jnp.float32),
            pltpu.SemaphoreType.DMA,
            pltpu.SemaphoreType.DMA,
        ],
    )
    def segsum(doc_hbm, tab_hbm, out_hbm, idx_v, rows_v, acc_v, s0, s1):
        wid = lax.axis_index("s") * NC + lax.axis_index("c")
        base = wid * TOK_PER_TILE
        sems = (s0, s1)

        pltpu.sync_copy(doc_hbm.at[pl.ds(base, TOK_PER_TILE)],
                        idx_v.at[pl.ds(0, TOK_PER_TILE)])

        def gathers(b, buf):
            sem = sems[buf]
            return (
                pltpu.make_async_copy(
                    tab_hbm.at[idx_v.at[pl.ds(b * L, CH0)]],
                    rows_v.at[buf, pl.ds(0, CH0)], sem),
                pltpu.make_async_copy(
                    tab_hbm.at[idx_v.at[pl.ds(b * L + CH0, CH1)]],
                    rows_v.at[buf, pl.ds(CH0, CH1)], sem),
            )

        def issue(b, buf):
            for g in gathers(b, buf):
                g.start()

        def drain(b, buf):
            for g in gathers(b, buf):
                g.wait()

        issue(0, 0)

        def per_doc(bb, _):
            for phase in range(2):
                b = 2 * bb + phase
                buf = phase

                @pl.when(b + 1 < DOCS_PER_TILE)
                def _prefetch():
                    issue(b + 1, 1 - buf)

                drain(b, buf)

                zero = jnp.zeros((16,), jnp.float32)

                def tok(t, accs):
                    new = []
                    for d in range(4):
                        new.append(accs[d] + rows_v[buf, t, pl.ds(16 * d, 16)])
                    return tuple(new)

                accs = lax.fori_loop(0, L, tok, (zero,) * 4)
                for d in range(4):
                    acc_v[pl.ds(b * DIM + 16 * d, 16)] = accs[d]
            return _

        lax.fori_loop(0, DOCS_PER_TILE // 2, per_doc, 0)
        pltpu.sync_copy(
            acc_v,
            out_hbm.at[pl.ds(wid * DOCS_PER_TILE * DIM, DOCS_PER_TILE * DIM)])

    return segsum(doc_flat, table)


def _tc_body(acc_ref, doc_ref, row0_ref, wd_ref, bd_ref, out_ref):
    npad = jnp.sum((doc_ref[...] == 0).astype(jnp.float32), axis=1,
                   keepdims=True)
    cnt = jnp.maximum(float(L) - npad, 1.0)
    enc = (acc_ref[...] - npad * row0_ref[...]) / cnt
    out_ref[...] = jnp.dot(enc, wd_ref[...],
                           preferred_element_type=jnp.float32) + bd_ref[...]


def _tc_decode(acc, doc, row0, Wd, bd2):
    bm = 512
    grid = B // bm
    return pl.pallas_call(
        _tc_body,
        grid=(grid,),
        in_specs=[
            pl.BlockSpec((bm, DIM), lambda i: (i, 0)),
            pl.BlockSpec((bm, L), lambda i: (i, 0)),
            pl.BlockSpec((1, DIM), lambda i: (0, 0)),
            pl.BlockSpec((DIM, NLAB), lambda i: (0, 0)),
            pl.BlockSpec((1, NLAB), lambda i: (0, 0)),
        ],
        out_specs=pl.BlockSpec((bm, NLAB), lambda i: (i, 0)),
        out_shape=jax.ShapeDtypeStruct((B, NLAB), jnp.float32),
    )(acc, doc, row0, Wd, bd2)


def kernel(doc, table, Wd, bd):
    acc_flat = _sc_segsum(doc.reshape(B * L), table)
    acc = acc_flat.reshape(B, DIM)
    row0 = lax.slice(table, (0, 0), (1, DIM))
    return _tc_decode(acc, doc, row0, Wd, bd.reshape(1, NLAB))
